# initial kernel scaffold (unmeasured)
import jax
import jax.numpy as jnp
from jax import lax
from jax.experimental import pallas as pl
from jax.experimental.pallas import tpu as pltpu

N_TOK = 4096
HALF = N_TOK // 2
N_CHUNKS = 1
ROWS = HALF // N_CHUNKS


def kernel(ids, E):
    v_local, d = E.shape

    my_x = lax.axis_index("x")
    my_y = lax.axis_index("y")

    ids_half = lax.dynamic_slice(ids, (my_x * HALF,), (HALF,))
    local = ids_half - my_y * v_local
    in_range = (local >= 0) & (local < v_local)
    safe = jnp.where(in_range, local, 0)
    part = jnp.take(E, safe, axis=0) * in_range[:, None].astype(E.dtype)

    def body(q_ref, out_ref, comm_ref, ysend, yrecv, xsend, xrecv):
        mx = lax.axis_index("x")
        my = lax.axis_index("y")

        barrier = pltpu.get_barrier_semaphore()
        for nbr in ((mx, 1 - my), (1 - mx, my)):
            pl.semaphore_signal(
                barrier, inc=1, device_id=nbr,
                device_id_type=pl.DeviceIdType.MESH,
            )
        pl.semaphore_wait(barrier, 2)

        y_rdmas = []
        for c in range(N_CHUNKS):
            sl = pl.ds(c * ROWS, ROWS)
            rdma = pltpu.make_async_remote_copy(
                src_ref=q_ref.at[sl],
                dst_ref=comm_ref.at[sl],
                send_sem=ysend.at[c],
                recv_sem=yrecv.at[c],
                device_id=(mx, 1 - my),
                device_id_type=pl.DeviceIdType.MESH,
            )
            rdma.start()
            y_rdmas.append(rdma)

        base = mx * HALF
        x_rdmas = []
        for c in range(N_CHUNKS):
            y_rdmas[c].wait_recv()
            sl = pl.ds(c * ROWS, ROWS)
            osl = pl.ds(base + c * ROWS, ROWS)
            out_ref[osl, :] = q_ref[sl, :] + comm_ref[sl, :]
            rdma = pltpu.make_async_remote_copy(
                src_ref=out_ref.at[osl],
                dst_ref=out_ref.at[osl],
                send_sem=xsend.at[c],
                recv_sem=xrecv.at[c],
                device_id=(1 - mx, my),
                device_id_type=pl.DeviceIdType.MESH,
            )
            rdma.start()
            x_rdmas.append(rdma)

        for c in range(N_CHUNKS):
            x_rdmas[c].wait_recv()
        for c in range(N_CHUNKS):
            y_rdmas[c].wait_send()
            x_rdmas[c].wait_send()

    return pl.pallas_call(
        body,
        out_shape=jax.ShapeDtypeStruct((N_TOK, d), jnp.float32),
        in_specs=[pl.BlockSpec(memory_space=pltpu.VMEM)],
        out_specs=pl.BlockSpec(memory_space=pltpu.VMEM),
        scratch_shapes=[
            pltpu.VMEM((HALF, d), jnp.float32),
            pltpu.SemaphoreType.DMA((N_CHUNKS,)),
            pltpu.SemaphoreType.DMA((N_CHUNKS,)),
            pltpu.SemaphoreType.DMA((N_CHUNKS,)),
            pltpu.SemaphoreType.DMA((N_CHUNKS,)),
        ],
        compiler_params=pltpu.CompilerParams(collective_id=0),
    )(part)


# baseline (device time: 1325343 ns/iter reference)
import jax
import jax.numpy as jnp
from jax import lax
from jax.experimental import pallas as pl
from jax.experimental.pallas import tpu as pltpu

N_TOK = 4096
HALF = N_TOK // 2
N_CHUNKS = 1
ROWS = HALF // N_CHUNKS


def kernel(ids, E):
    v_local, d = E.shape

    my_x = lax.axis_index("x")
    my_y = lax.axis_index("y")

    ids_half = lax.dynamic_slice(ids, (my_x * HALF,), (HALF,))
    local = ids_half - my_y * v_local
    in_range = (local >= 0) & (local < v_local)
    safe = jnp.where(in_range, local, 0)
    part = jnp.take(E, safe, axis=0) * in_range[:, None].astype(E.dtype)

    def body(q_ref, out_ref, comm_ref, xland_ref,
             ysend, yrecv, xsend, xrecv, mine_sems, other_sems):
        mx = lax.axis_index("x")
        my = lax.axis_index("y")

        barrier = pltpu.get_barrier_semaphore()
        for nbr in ((mx, 1 - my), (1 - mx, my)):
            pl.semaphore_signal(
                barrier, inc=1, device_id=nbr,
                device_id_type=pl.DeviceIdType.MESH,
            )
        pl.semaphore_wait(barrier, 2)

        y_rdmas = []
        for c in range(N_CHUNKS):
            sl = pl.ds(c * ROWS, ROWS)
            rdma = pltpu.make_async_remote_copy(
                src_ref=q_ref.at[sl],
                dst_ref=comm_ref.at[sl],
                send_sem=ysend.at[c],
                recv_sem=yrecv.at[c],
                device_id=(mx, 1 - my),
                device_id_type=pl.DeviceIdType.MESH,
            )
            rdma.start()
            y_rdmas.append(rdma)

        base = mx * HALF
        x_rdmas = []
        copies = []
        for c in range(N_CHUNKS):
            y_rdmas[c].wait_recv()
            sl = pl.ds(c * ROWS, ROWS)
            comm_ref[sl, :] = q_ref[sl, :] + comm_ref[sl, :]
            rdma = pltpu.make_async_remote_copy(
                src_ref=comm_ref.at[sl],
                dst_ref=xland_ref.at[sl],
                send_sem=xsend.at[c],
                recv_sem=xrecv.at[c],
                device_id=(1 - mx, my),
                device_id_type=pl.DeviceIdType.MESH,
            )
            rdma.start()
            x_rdmas.append(rdma)
            cp = pltpu.make_async_copy(
                comm_ref.at[sl],
                out_ref.at[pl.ds(base + c * ROWS, ROWS)],
                mine_sems.at[c],
            )
            cp.start()
            copies.append(cp)

        other = (1 - mx) * HALF
        for c in range(N_CHUNKS):
            x_rdmas[c].wait_recv()
            sl = pl.ds(c * ROWS, ROWS)
            cp = pltpu.make_async_copy(
                xland_ref.at[sl],
                out_ref.at[pl.ds(other + c * ROWS, ROWS)],
                other_sems.at[c],
            )
            cp.start()
            copies.append(cp)

        for cp in copies:
            cp.wait()
        for c in range(N_CHUNKS):
            y_rdmas[c].wait_send()
            x_rdmas[c].wait_send()

    return pl.pallas_call(
        body,
        out_shape=jax.ShapeDtypeStruct((N_TOK, d), jnp.float32),
        in_specs=[pl.BlockSpec(memory_space=pltpu.VMEM)],
        out_specs=pl.BlockSpec(memory_space=pl.ANY),
        scratch_shapes=[
            pltpu.VMEM((HALF, d), jnp.float32),
            pltpu.VMEM((HALF, d), jnp.float32),
            pltpu.SemaphoreType.DMA((N_CHUNKS,)),
            pltpu.SemaphoreType.DMA((N_CHUNKS,)),
            pltpu.SemaphoreType.DMA((N_CHUNKS,)),
            pltpu.SemaphoreType.DMA((N_CHUNKS,)),
            pltpu.SemaphoreType.DMA((N_CHUNKS,)),
            pltpu.SemaphoreType.DMA((N_CHUNKS,)),
        ],
        compiler_params=pltpu.CompilerParams(
            collective_id=0,
            vmem_limit_bytes=100 * 1024 * 1024,
        ),
    )(part)


# device time: 317751 ns/iter; 4.1710x vs baseline; 4.1710x over previous
import jax
import jax.numpy as jnp
from jax import lax
from jax.experimental import pallas as pl
from jax.experimental.pallas import tpu as pltpu

N_TOK = 4096
HALF = N_TOK // 2
N_CHUNKS = 16
ROWS = HALF // N_CHUNKS


def kernel(ids, E):
    v_local, d = E.shape

    my_x = lax.axis_index("x")
    my_y = lax.axis_index("y")

    ids_half = lax.dynamic_slice(ids, (my_x * HALF,), (HALF,))
    local = ids_half - my_y * v_local
    iota = lax.broadcasted_iota(jnp.int32, (HALF, v_local), 1)
    onehot = (local[:, None] == iota).astype(jnp.bfloat16)
    part = jnp.dot(onehot, E.astype(jnp.bfloat16),
                   preferred_element_type=jnp.float32)

    def body(q_ref, out_ref, comm_ref, xland_ref,
             ysend, yrecv, xsend, xrecv, mine_sems, other_sems):
        mx = lax.axis_index("x")
        my = lax.axis_index("y")

        barrier = pltpu.get_barrier_semaphore()
        for nbr in ((mx, 1 - my), (1 - mx, my)):
            pl.semaphore_signal(
                barrier, inc=1, device_id=nbr,
                device_id_type=pl.DeviceIdType.MESH,
            )
        pl.semaphore_wait(barrier, 2)

        y_rdmas = []
        for c in range(N_CHUNKS):
            sl = pl.ds(c * ROWS, ROWS)
            rdma = pltpu.make_async_remote_copy(
                src_ref=q_ref.at[sl],
                dst_ref=comm_ref.at[sl],
                send_sem=ysend.at[c],
                recv_sem=yrecv.at[c],
                device_id=(mx, 1 - my),
                device_id_type=pl.DeviceIdType.MESH,
            )
            rdma.start()
            y_rdmas.append(rdma)

        base = mx * HALF
        x_rdmas = []
        copies = []
        for c in range(N_CHUNKS):
            y_rdmas[c].wait_recv()
            sl = pl.ds(c * ROWS, ROWS)
            comm_ref[sl, :] = q_ref[sl, :] + comm_ref[sl, :]
            rdma = pltpu.make_async_remote_copy(
                src_ref=comm_ref.at[sl],
                dst_ref=xland_ref.at[sl],
                send_sem=xsend.at[c],
                recv_sem=xrecv.at[c],
                device_id=(1 - mx, my),
                device_id_type=pl.DeviceIdType.MESH,
            )
            rdma.start()
            x_rdmas.append(rdma)
            cp = pltpu.make_async_copy(
                comm_ref.at[sl],
                out_ref.at[pl.ds(base + c * ROWS, ROWS)],
                mine_sems.at[c],
            )
            cp.start()
            copies.append(cp)

        other = (1 - mx) * HALF
        for c in range(N_CHUNKS):
            x_rdmas[c].wait_recv()
            sl = pl.ds(c * ROWS, ROWS)
            cp = pltpu.make_async_copy(
                xland_ref.at[sl],
                out_ref.at[pl.ds(other + c * ROWS, ROWS)],
                other_sems.at[c],
            )
            cp.start()
            copies.append(cp)

        for cp in copies:
            cp.wait()
        for c in range(N_CHUNKS):
            y_rdmas[c].wait_send()
            x_rdmas[c].wait_send()

    return pl.pallas_call(
        body,
        out_shape=jax.ShapeDtypeStruct((N_TOK, d), jnp.float32),
        in_specs=[pl.BlockSpec(memory_space=pltpu.VMEM)],
        out_specs=pl.BlockSpec(memory_space=pl.ANY),
        scratch_shapes=[
            pltpu.VMEM((HALF, d), jnp.float32),
            pltpu.VMEM((HALF, d), jnp.float32),
            pltpu.SemaphoreType.DMA((N_CHUNKS,)),
            pltpu.SemaphoreType.DMA((N_CHUNKS,)),
            pltpu.SemaphoreType.DMA((N_CHUNKS,)),
            pltpu.SemaphoreType.DMA((N_CHUNKS,)),
            pltpu.SemaphoreType.DMA((N_CHUNKS,)),
            pltpu.SemaphoreType.DMA((N_CHUNKS,)),
        ],
        compiler_params=pltpu.CompilerParams(
            collective_id=0,
            vmem_limit_bytes=100 * 1024 * 1024,
        ),
    )(part)


# device time: 316918 ns/iter; 4.1820x vs baseline; 1.0026x over previous
import jax
import jax.numpy as jnp
from jax import lax
from jax.experimental import pallas as pl
from jax.experimental.pallas import tpu as pltpu

N_TOK = 4096
HALF = N_TOK // 2
N_CHUNKS = 16
ROWS = HALF // N_CHUNKS


def kernel(ids, E):
    v_local, d = E.shape

    my_x = lax.axis_index("x")
    my_y = lax.axis_index("y")

    ids_half = lax.dynamic_slice(ids, (my_x * HALF,), (HALF,))
    local = ids_half - my_y * v_local
    iota = lax.broadcasted_iota(jnp.int32, (HALF, v_local), 1)
    onehot = (local[:, None] == iota).astype(jnp.bfloat16)
    part = jnp.dot(onehot, E.astype(jnp.bfloat16),
                   preferred_element_type=jnp.float32)

    def body(q_ref, out_ref, comm_ref, ysend, yrecv, xsend, xrecv, cpm_sems):
        mx = lax.axis_index("x")
        my = lax.axis_index("y")

        barrier = pltpu.get_barrier_semaphore()
        for nbr in ((mx, 1 - my), (1 - mx, my)):
            pl.semaphore_signal(
                barrier, inc=1, device_id=nbr,
                device_id_type=pl.DeviceIdType.MESH,
            )
        pl.semaphore_wait(barrier, 2)

        y_rdmas = []
        for c in range(N_CHUNKS):
            sl = pl.ds(c * ROWS, ROWS)
            rdma = pltpu.make_async_remote_copy(
                src_ref=q_ref.at[sl],
                dst_ref=comm_ref.at[sl],
                send_sem=ysend.at[c],
                recv_sem=yrecv.at[c],
                device_id=(mx, 1 - my),
                device_id_type=pl.DeviceIdType.MESH,
            )
            rdma.start()
            y_rdmas.append(rdma)

        base = mx * HALF
        x_rdmas = []
        copies = []
        for c in range(N_CHUNKS):
            y_rdmas[c].wait_recv()
            sl = pl.ds(c * ROWS, ROWS)
            osl = pl.ds(base + c * ROWS, ROWS)
            comm_ref[sl, :] = q_ref[sl, :] + comm_ref[sl, :]
            rdma = pltpu.make_async_remote_copy(
                src_ref=comm_ref.at[sl],
                dst_ref=out_ref.at[osl],
                send_sem=xsend.at[c],
                recv_sem=xrecv.at[c],
                device_id=(1 - mx, my),
                device_id_type=pl.DeviceIdType.MESH,
            )
            rdma.start()
            x_rdmas.append(rdma)
            cp = pltpu.make_async_copy(
                comm_ref.at[sl],
                out_ref.at[osl],
                cpm_sems.at[c],
            )
            cp.start()
            copies.append(cp)

        for c in range(N_CHUNKS):
            x_rdmas[c].wait_recv()
        for cp in copies:
            cp.wait()
        for c in range(N_CHUNKS):
            y_rdmas[c].wait_send()
            x_rdmas[c].wait_send()

    return pl.pallas_call(
        body,
        out_shape=jax.ShapeDtypeStruct((N_TOK, d), jnp.float32),
        in_specs=[pl.BlockSpec(memory_space=pltpu.VMEM)],
        out_specs=pl.BlockSpec(memory_space=pl.ANY),
        scratch_shapes=[
            pltpu.VMEM((HALF, d), jnp.float32),
            pltpu.SemaphoreType.DMA((N_CHUNKS,)),
            pltpu.SemaphoreType.DMA((N_CHUNKS,)),
            pltpu.SemaphoreType.DMA((N_CHUNKS,)),
            pltpu.SemaphoreType.DMA((N_CHUNKS,)),
            pltpu.SemaphoreType.DMA((N_CHUNKS,)),
        ],
        compiler_params=pltpu.CompilerParams(
            collective_id=0,
            vmem_limit_bytes=100 * 1024 * 1024,
        ),
    )(part)


# device time: 272371 ns/iter; 4.8659x vs baseline; 1.1636x over previous
import jax
import jax.numpy as jnp
from jax import lax
from jax.experimental import pallas as pl
from jax.experimental.pallas import tpu as pltpu

N_TOK = 4096
HALF = N_TOK // 2
N_CHUNKS = 16
ROWS = HALF // N_CHUNKS
QR = 4


def kernel(ids, E):
    v_local, d = E.shape

    my_x = lax.axis_index("x")
    my_y = lax.axis_index("y")

    ids_half = lax.dynamic_slice(ids, (my_x * HALF,), (HALF,))
    local = (ids_half - my_y * v_local).reshape(HALF, 1)
    e_bf16 = E.astype(jnp.bfloat16)

    def body(local_ref, e_ref, out_ref, ring_ref, comm_ref,
             ysend, yrecv, xsend, xrecv, cpm_sems):
        mx = lax.axis_index("x")
        my = lax.axis_index("y")
        base = mx * HALF

        barrier = pltpu.get_barrier_semaphore()
        for nbr in ((mx, 1 - my), (1 - mx, my)):
            pl.semaphore_signal(
                barrier, inc=1, device_id=nbr,
                device_id_type=pl.DeviceIdType.MESH,
            )
        pl.semaphore_wait(barrier, 2)

        def y_rdma(c):
            return pltpu.make_async_remote_copy(
                src_ref=ring_ref.at[lax.rem(c, QR)],
                dst_ref=comm_ref.at[pl.ds(c * ROWS, ROWS)],
                send_sem=ysend.at[c],
                recv_sem=yrecv.at[c],
                device_id=(mx, 1 - my),
                device_id_type=pl.DeviceIdType.MESH,
            )

        def x_rdma(c):
            return pltpu.make_async_remote_copy(
                src_ref=comm_ref.at[pl.ds(c * ROWS, ROWS)],
                dst_ref=out_ref.at[pl.ds(base + c * ROWS, ROWS)],
                send_sem=xsend.at[c],
                recv_sem=xrecv.at[c],
                device_id=(1 - mx, my),
                device_id_type=pl.DeviceIdType.MESH,
            )

        def out_copy(c):
            return pltpu.make_async_copy(
                comm_ref.at[pl.ds(c * ROWS, ROWS)],
                out_ref.at[pl.ds(base + c * ROWS, ROWS)],
                cpm_sems.at[c],
            )

        def step(c, carry):
            @pl.when(c < N_CHUNKS)
            def _():
                slot = lax.rem(c, QR)
                idx = local_ref[pl.ds(c * ROWS, ROWS), :]
                iota = lax.broadcasted_iota(jnp.int32, (ROWS, v_local), 1)
                oh = (idx == iota).astype(jnp.bfloat16)
                ring_ref[slot] = jnp.dot(
                    oh, e_ref[:, :], preferred_element_type=jnp.float32)
                y_rdma(c).start()

            @pl.when(c >= 1)
            def _():
                j = c - 1
                rdma = y_rdma(j)
                rdma.wait_recv()
                sl = pl.ds(j * ROWS, ROWS)
                comm_ref[sl, :] = ring_ref[lax.rem(j, QR)] + comm_ref[sl, :]
                rdma.wait_send()
                x_rdma(j).start()
                out_copy(j).start()

            return carry

        lax.fori_loop(0, N_CHUNKS + 1, step, 0)

        def drain(j, carry):
            rdma = x_rdma(j)
            rdma.wait_recv()
            out_copy(j).wait()
            rdma.wait_send()
            return carry

        lax.fori_loop(0, N_CHUNKS, drain, 0)

    return pl.pallas_call(
        body,
        out_shape=jax.ShapeDtypeStruct((N_TOK, d), jnp.float32),
        in_specs=[
            pl.BlockSpec(memory_space=pltpu.VMEM),
            pl.BlockSpec(memory_space=pltpu.VMEM),
        ],
        out_specs=pl.BlockSpec(memory_space=pl.ANY),
        scratch_shapes=[
            pltpu.VMEM((QR, ROWS, d), jnp.float32),
            pltpu.VMEM((HALF, d), jnp.float32),
            pltpu.SemaphoreType.DMA((N_CHUNKS,)),
            pltpu.SemaphoreType.DMA((N_CHUNKS,)),
            pltpu.SemaphoreType.DMA((N_CHUNKS,)),
            pltpu.SemaphoreType.DMA((N_CHUNKS,)),
            pltpu.SemaphoreType.DMA((N_CHUNKS,)),
        ],
        compiler_params=pltpu.CompilerParams(
            collective_id=0,
            vmem_limit_bytes=100 * 1024 * 1024,
        ),
    )(local, e_bf16)


# device time: 272321 ns/iter; 4.8668x vs baseline; 1.0002x over previous
import jax
import jax.numpy as jnp
from jax import lax
from jax.experimental import pallas as pl
from jax.experimental.pallas import tpu as pltpu

N_TOK = 4096
HALF = N_TOK // 2
N_CHUNKS = 16
ROWS = HALF // N_CHUNKS
QR = 4


def kernel(ids, E):
    v_local, d = E.shape

    my_x = lax.axis_index("x")
    my_y = lax.axis_index("y")

    ids_half = lax.dynamic_slice(ids, (my_x * HALF,), (HALF,))
    local = (ids_half - my_y * v_local).reshape(HALF, 1)
    e_bf16 = E.astype(jnp.bfloat16)

    def body(local_ref, e_ref, out_ref, ring_ref, comm_ref,
             ysend, yrecv, xsend, xrecv, cpm_sems):
        mx = lax.axis_index("x")
        my = lax.axis_index("y")
        base = mx * HALF

        barrier = pltpu.get_barrier_semaphore()
        for nbr in ((mx, 1 - my), (1 - mx, my)):
            pl.semaphore_signal(
                barrier, inc=1, device_id=nbr,
                device_id_type=pl.DeviceIdType.MESH,
            )
        pl.semaphore_wait(barrier, 2)

        def chunk_ds(c, off=0):
            return pl.ds(pl.multiple_of(off + c * ROWS, ROWS), ROWS)

        def y_rdma(c):
            return pltpu.make_async_remote_copy(
                src_ref=ring_ref.at[lax.rem(c, QR)],
                dst_ref=comm_ref.at[chunk_ds(c)],
                send_sem=ysend.at[c],
                recv_sem=yrecv.at[c],
                device_id=(mx, 1 - my),
                device_id_type=pl.DeviceIdType.MESH,
            )

        def x_rdma(c):
            return pltpu.make_async_remote_copy(
                src_ref=comm_ref.at[chunk_ds(c)],
                dst_ref=out_ref.at[chunk_ds(c, base)],
                send_sem=xsend.at[c],
                recv_sem=xrecv.at[c],
                device_id=(1 - mx, my),
                device_id_type=pl.DeviceIdType.MESH,
            )

        def out_copy(c):
            return pltpu.make_async_copy(
                comm_ref.at[chunk_ds(c)],
                out_ref.at[chunk_ds(c, base)],
                cpm_sems.at[c],
            )

        def step(c, carry):
            @pl.when(c < N_CHUNKS)
            def _():
                @pl.when(c >= QR)
                def _():
                    y_rdma(c - QR).wait_send()

                slot = lax.rem(c, QR)
                idx = local_ref[chunk_ds(c), :]
                iota = lax.broadcasted_iota(jnp.int32, (ROWS, v_local), 1)
                oh = (idx == iota).astype(jnp.bfloat16)
                ring_ref[slot] = jnp.dot(
                    oh, e_ref[:, :], preferred_element_type=jnp.float32)
                y_rdma(c).start()

            @pl.when(c >= 1)
            def _():
                j = c - 1
                y_rdma(j).wait_recv()
                sl = chunk_ds(j)
                comm_ref[sl, :] = ring_ref[lax.rem(j, QR)] + comm_ref[sl, :]
                x_rdma(j).start()
                out_copy(j).start()

            return carry

        lax.fori_loop(0, N_CHUNKS + 1, step, 0)

        def drain(j, carry):
            rdma = x_rdma(j)
            rdma.wait_recv()
            out_copy(j).wait()
            rdma.wait_send()
            @pl.when(j >= N_CHUNKS - QR)
            def _():
                y_rdma(j).wait_send()
            return carry

        lax.fori_loop(0, N_CHUNKS, drain, 0)

    return pl.pallas_call(
        body,
        out_shape=jax.ShapeDtypeStruct((N_TOK, d), jnp.float32),
        in_specs=[
            pl.BlockSpec(memory_space=pltpu.VMEM),
            pl.BlockSpec(memory_space=pltpu.VMEM),
        ],
        out_specs=pl.BlockSpec(memory_space=pl.ANY),
        scratch_shapes=[
            pltpu.VMEM((QR, ROWS, d), jnp.float32),
            pltpu.VMEM((HALF, d), jnp.float32),
            pltpu.SemaphoreType.DMA((N_CHUNKS,)),
            pltpu.SemaphoreType.DMA((N_CHUNKS,)),
            pltpu.SemaphoreType.DMA((N_CHUNKS,)),
            pltpu.SemaphoreType.DMA((N_CHUNKS,)),
            pltpu.SemaphoreType.DMA((N_CHUNKS,)),
        ],
        compiler_params=pltpu.CompilerParams(
            collective_id=0,
            vmem_limit_bytes=100 * 1024 * 1024,
        ),
    )(local, e_bf16)
